# baseline (speedup 1.0000x reference)
import functools

import jax
import jax.numpy as jnp
from jax import lax
from jax.experimental import pallas as pl
from jax.experimental.pallas import tpu as pltpu
from jax.experimental.pallas import tpu_sc as plsc

_NC = 2
_NS = 16
_NW = _NC * _NS
_CH = 125


def _matmul1(x, w):
    def body(x_ref, w_ref, o_ref):
        o_ref[...] = jnp.dot(x_ref[...], w_ref[...],
                             preferred_element_type=jnp.float32)
    return pl.pallas_call(
        body,
        out_shape=jax.ShapeDtypeStruct((x.shape[0], w.shape[1]), jnp.float32),
    )(x, w)


def _sc_aggregate(h, src2, dst2, zeros):
    n, f = zeros.shape
    n_chunks = src2.shape[0]
    chunks_per_w = n_chunks // _NW
    rows_per_tile = n // _NS

    mesh = plsc.VectorSubcoreMesh(core_axis_name="c", subcore_axis_name="s")

    @functools.partial(
        pl.kernel,
        mesh=mesh,
        out_type=jax.ShapeDtypeStruct((_NC, n, f), jnp.float32),
        scratch_types=[
            pltpu.VMEM((chunks_per_w, _CH), jnp.int32),
            pltpu.VMEM((chunks_per_w, _CH), jnp.int32),
            pltpu.VMEM((_CH, f), jnp.float32),
            pltpu.VMEM_SHARED((n, f), jnp.float32),
            pltpu.SemaphoreType.DMA,
        ],
        compiler_params=pltpu.CompilerParams(use_tc_tiling_on_sc=False),
    )
    def agg(h_hbm, src_hbm, dst_hbm, z_hbm, out_hbm,
            sidx_v, didx_v, rows_v, acc_sh, sem):
        cid = lax.axis_index("c")
        sid = lax.axis_index("s")
        wid = sid * _NC + cid

        pltpu.sync_copy(z_hbm.at[pl.ds(sid * rows_per_tile, rows_per_tile)],
                        acc_sh.at[pl.ds(sid * rows_per_tile, rows_per_tile)])
        pltpu.sync_copy(src_hbm.at[pl.ds(wid * chunks_per_w, chunks_per_w)],
                        sidx_v)
        pltpu.sync_copy(dst_hbm.at[pl.ds(wid * chunks_per_w, chunks_per_w)],
                        didx_v)
        plsc.subcore_barrier()

        def step(j, carry):
            pltpu.async_copy(h_hbm.at[sidx_v.at[j]], rows_v, sem).wait()
            pltpu.sync_copy(rows_v, acc_sh.at[didx_v.at[j]], add=True)
            return carry

        lax.fori_loop(0, chunks_per_w, step, 0)
        plsc.subcore_barrier()

        pltpu.sync_copy(acc_sh.at[pl.ds(sid * rows_per_tile, rows_per_tile)],
                        out_hbm.at[cid, pl.ds(sid * rows_per_tile,
                                              rows_per_tile)])

    return agg(h, src2, dst2, zeros)


def _layer2_matmul(partial, w):
    def body(p_ref, w_ref, o_ref):
        hidden = jnp.maximum(p_ref[0] + p_ref[1], 0.0)
        o_ref[...] = jnp.dot(hidden, w_ref[...],
                             preferred_element_type=jnp.float32)
    n_pad = partial.shape[1]
    return pl.pallas_call(
        body,
        out_shape=jax.ShapeDtypeStruct((n_pad, w.shape[1]), jnp.float32),
    )(partial, w)


def _prep_z(partial, n):
    f = partial.shape[2]

    def body(p_ref, z_ref, zt_ref):
        z = p_ref[0, :n, :] + p_ref[1, :n, :]
        z_ref[...] = z
        zt_ref[...] = z.T

    return pl.pallas_call(
        body,
        out_shape=(jax.ShapeDtypeStruct((n, f), jnp.float32),
                   jax.ShapeDtypeStruct((f, n), jnp.float32)),
    )(partial)


def _decoder(z, zt, n):
    f = z.shape[1]
    blk = 400
    grid = n // blk

    def body(zb_ref, zt_ref, o_ref):
        o_ref[...] = jnp.dot(zb_ref[...], zt_ref[...],
                             preferred_element_type=jnp.float32)

    return pl.pallas_call(
        body,
        grid=(grid,),
        in_specs=[pl.BlockSpec((blk, f), lambda i: (i, 0)),
                  pl.BlockSpec((f, n), lambda i: (0, 0))],
        out_specs=pl.BlockSpec((blk, n), lambda i: (i, 0)),
        out_shape=jax.ShapeDtypeStruct((n, n), jnp.float32),
    )(z, zt)


def kernel(x, edge_index, W1, W2):
    n = x.shape[0]
    e = edge_index.shape[1]
    n_pad = ((n // _NS + 7) // 8 * 8) * _NS
    src2 = edge_index[0].astype(jnp.int32).reshape(e // _CH, _CH)
    dst2 = edge_index[1].astype(jnp.int32).reshape(e // _CH, _CH)
    zeros1 = jnp.zeros((n_pad, W1.shape[1]), jnp.float32)
    zeros2 = jnp.zeros((n_pad, W2.shape[1]), jnp.float32)

    h = _matmul1(x, W1)
    p1 = _sc_aggregate(h, src2, dst2, zeros1)
    h2 = _layer2_matmul(p1, W2)
    p2 = _sc_aggregate(h2, src2, dst2, zeros2)
    z, zt = _prep_z(p2, n)
    return _decoder(z, zt, n).reshape(n * n)


# speedup vs baseline: 1.0029x; 1.0029x over previous
import functools

import jax
import jax.numpy as jnp
from jax import lax
from jax.experimental import pallas as pl
from jax.experimental.pallas import tpu as pltpu
from jax.experimental.pallas import tpu_sc as plsc

_NC = 2
_NS = 16
_NW = _NC * _NS
_CH = 125


def _matmul1(x, w):
    def body(x_ref, w_ref, o_ref):
        o_ref[...] = jnp.dot(x_ref[...], w_ref[...],
                             preferred_element_type=jnp.float32)
    return pl.pallas_call(
        body,
        out_shape=jax.ShapeDtypeStruct((x.shape[0], w.shape[1]), jnp.float32),
    )(x, w)


def _sc_aggregate(h, src2, dst2, zeros):
    n, f = zeros.shape
    n_chunks = src2.shape[0]
    chunks_per_w = n_chunks // _NW
    rows_per_tile = n // _NS

    mesh = plsc.VectorSubcoreMesh(core_axis_name="c", subcore_axis_name="s")

    @functools.partial(
        pl.kernel,
        mesh=mesh,
        out_type=jax.ShapeDtypeStruct((_NC, n, f), jnp.float32),
        scratch_types=[
            pltpu.VMEM((chunks_per_w, _CH), jnp.int32),
            pltpu.VMEM((chunks_per_w, _CH), jnp.int32),
            pltpu.VMEM((_CH, f), jnp.float32),
            pltpu.VMEM_SHARED((n, f), jnp.float32),
            pltpu.SemaphoreType.DMA,
        ],
        compiler_params=pltpu.CompilerParams(use_tc_tiling_on_sc=False),
    )
    def agg(h_hbm, src_hbm, dst_hbm, z_hbm, out_hbm,
            sidx_v, didx_v, rows_v, acc_sh, sem):
        cid = lax.axis_index("c")
        sid = lax.axis_index("s")
        wid = sid * _NC + cid

        pltpu.sync_copy(z_hbm.at[pl.ds(sid * rows_per_tile, rows_per_tile)],
                        acc_sh.at[pl.ds(sid * rows_per_tile, rows_per_tile)])
        pltpu.sync_copy(src_hbm.at[pl.ds(wid * chunks_per_w, chunks_per_w)],
                        sidx_v)
        pltpu.sync_copy(dst_hbm.at[pl.ds(wid * chunks_per_w, chunks_per_w)],
                        didx_v)
        plsc.subcore_barrier()

        def step(j, carry):
            pltpu.async_copy(h_hbm.at[sidx_v.at[j]], rows_v, sem).wait()
            pltpu.sync_copy(rows_v, acc_sh.at[didx_v.at[j]], add=True)
            return carry

        lax.fori_loop(0, chunks_per_w, step, 0)
        plsc.subcore_barrier()

        pltpu.sync_copy(acc_sh.at[pl.ds(sid * rows_per_tile, rows_per_tile)],
                        out_hbm.at[cid, pl.ds(sid * rows_per_tile,
                                              rows_per_tile)])

    return agg(h, src2, dst2, zeros)


def _layer2_matmul(partial, w):
    def body(p_ref, w_ref, o_ref):
        hidden = jnp.maximum(p_ref[0] + p_ref[1], 0.0)
        o_ref[...] = jnp.dot(hidden, w_ref[...],
                             preferred_element_type=jnp.float32)
    n_pad = partial.shape[1]
    return pl.pallas_call(
        body,
        out_shape=jax.ShapeDtypeStruct((n_pad, w.shape[1]), jnp.float32),
    )(partial, w)


def _prep_z(partial, n):
    f = partial.shape[2]

    def body(p_ref, z_ref, zt_ref):
        z = p_ref[0, :n, :] + p_ref[1, :n, :]
        z_ref[...] = z
        zt_ref[...] = z.T

    return pl.pallas_call(
        body,
        out_shape=(jax.ShapeDtypeStruct((n, f), jnp.float32),
                   jax.ShapeDtypeStruct((f, n), jnp.float32)),
    )(partial)


def _decoder(z, zt, n):
    f = z.shape[1]
    blk = 400

    def body(zb_ref, zt_ref, o_ref):
        o_ref[...] = jnp.dot(zb_ref[...], zt_ref[...],
                             preferred_element_type=jnp.float32)

    out2 = pl.pallas_call(
        body,
        grid=(n // blk,),
        in_specs=[pl.BlockSpec((blk, f), lambda i: (i, 0)),
                  pl.BlockSpec((f, n), lambda i: (0, 0))],
        out_specs=pl.BlockSpec((blk, n), lambda i: (i, 0)),
        out_shape=jax.ShapeDtypeStruct((n, n), jnp.float32),
    )(z, zt)
    return out2.reshape(-1)


def kernel(x, edge_index, W1, W2):
    n = x.shape[0]
    e = edge_index.shape[1]
    n_pad = ((n // _NS + 7) // 8 * 8) * _NS
    src2 = edge_index[0].astype(jnp.int32).reshape(e // _CH, _CH)
    dst2 = edge_index[1].astype(jnp.int32).reshape(e // _CH, _CH)
    zeros1 = jnp.zeros((n_pad, W1.shape[1]), jnp.float32)
    zeros2 = jnp.zeros((n_pad, W2.shape[1]), jnp.float32)

    h = _matmul1(x, W1)
    p1 = _sc_aggregate(h, src2, dst2, zeros1)
    h2 = _layer2_matmul(p1, W2)
    p2 = _sc_aggregate(h2, src2, dst2, zeros2)
    z, zt = _prep_z(p2, n)
    return _decoder(z, zt, n)


# speedup vs baseline: 1.0040x; 1.0011x over previous
import functools

import jax
import jax.numpy as jnp
from jax import lax
from jax.experimental import pallas as pl
from jax.experimental.pallas import tpu as pltpu
from jax.experimental.pallas import tpu_sc as plsc

_NC = 2
_NS = 16
_NW = _NC * _NS
_CH = 125


def _matmul1(x, w):
    def body(x_ref, w_ref, o_ref):
        o_ref[...] = jnp.dot(x_ref[...], w_ref[...],
                             preferred_element_type=jnp.float32)
    return pl.pallas_call(
        body,
        out_shape=jax.ShapeDtypeStruct((x.shape[0], w.shape[1]), jnp.float32),
    )(x, w)


def _sc_aggregate(h, src2, dst2, zeros):
    n, f = zeros.shape
    n_chunks = src2.shape[0]
    chunks_per_w = n_chunks // _NW
    rows_per_tile = n // _NS

    mesh = plsc.VectorSubcoreMesh(core_axis_name="c", subcore_axis_name="s")

    @functools.partial(
        pl.kernel,
        mesh=mesh,
        out_type=jax.ShapeDtypeStruct((_NC, n, f), jnp.float32),
        scratch_types=[
            pltpu.VMEM((chunks_per_w, _CH), jnp.int32),
            pltpu.VMEM((chunks_per_w, _CH), jnp.int32),
            pltpu.VMEM((_CH, f), jnp.float32),
            pltpu.VMEM_SHARED((n, f), jnp.float32),
            pltpu.SemaphoreType.DMA,
        ],
        compiler_params=pltpu.CompilerParams(use_tc_tiling_on_sc=False),
    )
    def agg(h_hbm, src_hbm, dst_hbm, z_hbm, out_hbm,
            sidx_v, didx_v, rows_v, acc_sh, sem):
        cid = lax.axis_index("c")
        sid = lax.axis_index("s")
        wid = sid * _NC + cid

        pltpu.sync_copy(z_hbm.at[pl.ds(sid * rows_per_tile, rows_per_tile)],
                        acc_sh.at[pl.ds(sid * rows_per_tile, rows_per_tile)])
        pltpu.sync_copy(src_hbm.at[pl.ds(wid * chunks_per_w, chunks_per_w)],
                        sidx_v)
        pltpu.sync_copy(dst_hbm.at[pl.ds(wid * chunks_per_w, chunks_per_w)],
                        didx_v)
        plsc.subcore_barrier()

        def step(j, carry):
            pltpu.async_copy(h_hbm.at[sidx_v.at[j]], rows_v, sem).wait()
            pltpu.sync_copy(rows_v, acc_sh.at[didx_v.at[j]], add=True)
            return carry

        lax.fori_loop(0, chunks_per_w, step, 0)
        plsc.subcore_barrier()

        pltpu.sync_copy(acc_sh.at[pl.ds(sid * rows_per_tile, rows_per_tile)],
                        out_hbm.at[cid, pl.ds(sid * rows_per_tile,
                                              rows_per_tile)])

    return agg(h, src2, dst2, zeros)


def _layer2_matmul(partial, w):
    def body(p_ref, w_ref, o_ref):
        hidden = jnp.maximum(p_ref[0] + p_ref[1], 0.0)
        o_ref[...] = jnp.dot(hidden, w_ref[...],
                             preferred_element_type=jnp.float32)
    n_pad = partial.shape[1]
    return pl.pallas_call(
        body,
        out_shape=jax.ShapeDtypeStruct((n_pad, w.shape[1]), jnp.float32),
    )(partial, w)


def _prep_z(partial, n):
    f = partial.shape[2]

    def body(p_ref, z_ref, zt_ref):
        z = p_ref[0, :n, :] + p_ref[1, :n, :]
        z_ref[...] = z
        zt_ref[...] = z.T

    return pl.pallas_call(
        body,
        out_shape=(jax.ShapeDtypeStruct((n, f), jnp.float32),
                   jax.ShapeDtypeStruct((f, n), jnp.float32)),
    )(partial)


def _decoder(z, zt, n):
    f = z.shape[1]
    blk = 400
    grid = n // blk

    def body(zb_ref, zt_ref, o_ref):
        o_ref[...] = jnp.dot(zb_ref[...], zt_ref[...],
                             preferred_element_type=jnp.float32)

    out2 = pl.pallas_call(
        body,
        grid=(grid,),
        in_specs=[pl.BlockSpec((blk, f), lambda i: (i, 0)),
                  pl.BlockSpec((f, n), lambda i: (0, 0))],
        out_specs=pl.BlockSpec((blk, n), lambda i: (i, 0)),
        out_shape=jax.ShapeDtypeStruct((n, n), jnp.float32),
    )(z, zt)
    return out2.reshape(-1)


def kernel(x, edge_index, W1, W2):
    n = x.shape[0]
    e = edge_index.shape[1]
    n_pad = ((n // _NS + 7) // 8 * 8) * _NS
    src2 = edge_index[0].astype(jnp.int32).reshape(e // _CH, _CH)
    dst2 = edge_index[1].astype(jnp.int32).reshape(e // _CH, _CH)
    zeros1 = jnp.zeros((n_pad, W1.shape[1]), jnp.float32)
    zeros2 = jnp.zeros((n_pad, W2.shape[1]), jnp.float32)

    h = _matmul1(x, W1)
    p1 = _sc_aggregate(h, src2, dst2, zeros1)
    h2 = _layer2_matmul(p1, W2)
    p2 = _sc_aggregate(h2, src2, dst2, zeros2)
    z, zt = _prep_z(p2, n)
    return _decoder(z, zt, n)


# speedup vs baseline: 1.0058x; 1.0018x over previous
import functools

import jax
import jax.numpy as jnp
from jax import lax
from jax.experimental import pallas as pl
from jax.experimental.pallas import tpu as pltpu
from jax.experimental.pallas import tpu_sc as plsc

_NC = 2
_NS = 16
_NW = _NC * _NS
_CH = 125


def _matmul1(x, w):
    def body(x_ref, w_ref, o_ref):
        o_ref[...] = jnp.dot(x_ref[...], w_ref[...],
                             preferred_element_type=jnp.float32)
    return pl.pallas_call(
        body,
        out_shape=jax.ShapeDtypeStruct((x.shape[0], w.shape[1]), jnp.float32),
    )(x, w)


def _sc_aggregate(h, src2, dst2, zeros):
    n, f = zeros.shape
    n_chunks = src2.shape[0]
    chunks_per_w = n_chunks // _NW
    rows_per_tile = n // _NS

    mesh = plsc.VectorSubcoreMesh(core_axis_name="c", subcore_axis_name="s")

    @functools.partial(
        pl.kernel,
        mesh=mesh,
        out_type=jax.ShapeDtypeStruct((_NC, n, f), jnp.float32),
        scratch_types=[
            pltpu.VMEM((chunks_per_w, _CH), jnp.int32),
            pltpu.VMEM((chunks_per_w, _CH), jnp.int32),
            pltpu.VMEM((_CH, f), jnp.float32),
            pltpu.VMEM_SHARED((n, f), jnp.float32),
            pltpu.SemaphoreType.DMA,
        ],
        compiler_params=pltpu.CompilerParams(use_tc_tiling_on_sc=False),
    )
    def agg(h_hbm, src_hbm, dst_hbm, z_hbm, out_hbm,
            sidx_v, didx_v, rows_v, acc_sh, sem):
        cid = lax.axis_index("c")
        sid = lax.axis_index("s")
        wid = sid * _NC + cid

        pltpu.sync_copy(z_hbm.at[pl.ds(sid * rows_per_tile, rows_per_tile)],
                        acc_sh.at[pl.ds(sid * rows_per_tile, rows_per_tile)])
        pltpu.sync_copy(src_hbm.at[pl.ds(wid * chunks_per_w, chunks_per_w)],
                        sidx_v)
        pltpu.sync_copy(dst_hbm.at[pl.ds(wid * chunks_per_w, chunks_per_w)],
                        didx_v)
        plsc.subcore_barrier()

        def step(j, carry):
            pltpu.async_copy(h_hbm.at[sidx_v.at[j]], rows_v, sem).wait()
            pltpu.sync_copy(rows_v, acc_sh.at[didx_v.at[j]], add=True)
            return carry

        lax.fori_loop(0, chunks_per_w, step, 0)
        plsc.subcore_barrier()

        pltpu.sync_copy(acc_sh.at[pl.ds(sid * rows_per_tile, rows_per_tile)],
                        out_hbm.at[cid, pl.ds(sid * rows_per_tile,
                                              rows_per_tile)])

    return agg(h, src2, dst2, zeros)


def _layer2_matmul(partial, w):
    def body(p_ref, w_ref, o_ref):
        hidden = jnp.maximum(p_ref[0] + p_ref[1], 0.0)
        o_ref[...] = jnp.dot(hidden, w_ref[...],
                             preferred_element_type=jnp.float32)
    n_pad = partial.shape[1]
    return pl.pallas_call(
        body,
        out_shape=jax.ShapeDtypeStruct((n_pad, w.shape[1]), jnp.float32),
    )(partial, w)


def _prep_z(partial, n):
    f = partial.shape[2]

    def body(p_ref, z_ref, zt_ref):
        z = (p_ref[0, :n, :] + p_ref[1, :n, :]).astype(jnp.bfloat16)
        z_ref[...] = z
        zt_ref[...] = z.T

    return pl.pallas_call(
        body,
        out_shape=(jax.ShapeDtypeStruct((n, f), jnp.bfloat16),
                   jax.ShapeDtypeStruct((f, n), jnp.bfloat16)),
    )(partial)


def _decoder(z, zt, n):
    f = z.shape[1]
    blk = 400
    grid = n // blk

    def body(zb_ref, zt_ref, o_ref):
        o_ref[...] = jnp.dot(zb_ref[...], zt_ref[...],
                             preferred_element_type=jnp.float32)

    out2 = pl.pallas_call(
        body,
        grid=(grid,),
        in_specs=[pl.BlockSpec((blk, f), lambda i: (i, 0)),
                  pl.BlockSpec((f, n), lambda i: (0, 0))],
        out_specs=pl.BlockSpec((blk, n), lambda i: (i, 0)),
        out_shape=jax.ShapeDtypeStruct((n, n), jnp.float32),
    )(z, zt)
    return out2.reshape(-1)


def kernel(x, edge_index, W1, W2):
    n = x.shape[0]
    e = edge_index.shape[1]
    n_pad = ((n // _NS + 7) // 8 * 8) * _NS
    src2 = edge_index[0].astype(jnp.int32).reshape(e // _CH, _CH)
    dst2 = edge_index[1].astype(jnp.int32).reshape(e // _CH, _CH)
    zeros1 = jnp.zeros((n_pad, W1.shape[1]), jnp.float32)
    zeros2 = jnp.zeros((n_pad, W2.shape[1]), jnp.float32)

    h = _matmul1(x, W1)
    p1 = _sc_aggregate(h, src2, dst2, zeros1)
    h2 = _layer2_matmul(p1, W2)
    p2 = _sc_aggregate(h2, src2, dst2, zeros2)
    z, zt = _prep_z(p2, n)
    return _decoder(z, zt, n)


# speedup vs baseline: 1.1102x; 1.1038x over previous
import functools

import jax
import jax.numpy as jnp
from jax import lax
from jax.experimental import pallas as pl
from jax.experimental.pallas import tpu as pltpu
from jax.experimental.pallas import tpu_sc as plsc

_NC = 2
_NS = 16
_NW = _NC * _NS
_CH = 125


def _matmul1(x, w):
    def body(x_ref, w_ref, o_ref):
        o_ref[...] = jnp.dot(x_ref[...], w_ref[...],
                             preferred_element_type=jnp.float32)
    return pl.pallas_call(
        body,
        out_shape=jax.ShapeDtypeStruct((x.shape[0], w.shape[1]), jnp.float32),
    )(x, w)


def _sc_aggregate(h, src2, dst2, zeros):
    n, f = zeros.shape
    n_chunks = src2.shape[0]
    chunks_per_w = n_chunks // _NW
    rows_per_tile = n // _NS

    mesh = plsc.VectorSubcoreMesh(core_axis_name="c", subcore_axis_name="s")

    @functools.partial(
        pl.kernel,
        mesh=mesh,
        out_type=jax.ShapeDtypeStruct((_NC, n, f), jnp.float32),
        scratch_types=[
            pltpu.VMEM((chunks_per_w, _CH), jnp.int32),
            pltpu.VMEM((chunks_per_w, _CH), jnp.int32),
            pltpu.VMEM((_CH, f), jnp.float32),
            pltpu.VMEM((_CH, f), jnp.float32),
            pltpu.VMEM_SHARED((n, f), jnp.float32),
            pltpu.SemaphoreType.DMA,
            pltpu.SemaphoreType.DMA,
        ],
        compiler_params=pltpu.CompilerParams(use_tc_tiling_on_sc=False),
    )
    def agg(h_hbm, src_hbm, dst_hbm, z_hbm, out_hbm,
            sidx_v, didx_v, rows0_v, rows1_v, acc_sh, sem0, sem1):
        cid = lax.axis_index("c")
        sid = lax.axis_index("s")
        wid = sid * _NC + cid

        pltpu.sync_copy(z_hbm.at[pl.ds(sid * rows_per_tile, rows_per_tile)],
                        acc_sh.at[pl.ds(sid * rows_per_tile, rows_per_tile)])
        pltpu.sync_copy(src_hbm.at[pl.ds(wid * chunks_per_w, chunks_per_w)],
                        sidx_v)
        pltpu.sync_copy(dst_hbm.at[pl.ds(wid * chunks_per_w, chunks_per_w)],
                        didx_v)
        plsc.subcore_barrier()

        pairs = chunks_per_w // 2

        def start(j, rows):
            sem = sem0 if rows is rows0_v else sem1
            pltpu.async_copy(h_hbm.at[sidx_v.at[j]], rows, sem)

        def wait_add(j, rows):
            sem = sem0 if rows is rows0_v else sem1
            pltpu.make_async_copy(h_hbm.at[sidx_v.at[j]], rows, sem).wait()
            pltpu.sync_copy(rows, acc_sh.at[didx_v.at[j]], add=True)

        start(0, rows0_v)

        def step(j, carry):
            a = 2 * j
            start(a + 1, rows1_v)
            wait_add(a, rows0_v)
            start(a + 2, rows0_v)
            wait_add(a + 1, rows1_v)
            return carry

        lax.fori_loop(0, pairs - 1, step, 0)
        last = 2 * (pairs - 1)
        start(last + 1, rows1_v)
        wait_add(last, rows0_v)
        wait_add(last + 1, rows1_v)
        plsc.subcore_barrier()

        pltpu.sync_copy(acc_sh.at[pl.ds(sid * rows_per_tile, rows_per_tile)],
                        out_hbm.at[cid, pl.ds(sid * rows_per_tile,
                                              rows_per_tile)])

    return agg(h, src2, dst2, zeros)


def _layer2_matmul(partial, w):
    def body(p_ref, w_ref, o_ref):
        hidden = jnp.maximum(p_ref[0] + p_ref[1], 0.0)
        o_ref[...] = jnp.dot(hidden, w_ref[...],
                             preferred_element_type=jnp.float32)
    n_pad = partial.shape[1]
    return pl.pallas_call(
        body,
        out_shape=jax.ShapeDtypeStruct((n_pad, w.shape[1]), jnp.float32),
    )(partial, w)


def _prep_z(partial, n):
    f = partial.shape[2]

    def body(p_ref, z_ref, zt_ref):
        z = p_ref[0, :n, :] + p_ref[1, :n, :]
        z_ref[...] = z
        zt_ref[...] = z.T

    return pl.pallas_call(
        body,
        out_shape=(jax.ShapeDtypeStruct((n, f), jnp.float32),
                   jax.ShapeDtypeStruct((f, n), jnp.float32)),
    )(partial)


def _decoder(z, zt, n):
    f = z.shape[1]
    blk = 400
    grid = n // blk

    def body(zb_ref, zt_ref, o_ref):
        o_ref[...] = jnp.dot(zb_ref[...], zt_ref[...],
                             preferred_element_type=jnp.float32)

    out2 = pl.pallas_call(
        body,
        grid=(grid,),
        in_specs=[pl.BlockSpec((blk, f), lambda i: (i, 0)),
                  pl.BlockSpec((f, n), lambda i: (0, 0))],
        out_specs=pl.BlockSpec((blk, n), lambda i: (i, 0)),
        out_shape=jax.ShapeDtypeStruct((n, n), jnp.float32),
    )(z, zt)
    return out2.reshape(-1)


def kernel(x, edge_index, W1, W2):
    n = x.shape[0]
    e = edge_index.shape[1]
    n_pad = ((n // _NS + 7) // 8 * 8) * _NS
    src2 = edge_index[0].astype(jnp.int32).reshape(e // _CH, _CH)
    dst2 = edge_index[1].astype(jnp.int32).reshape(e // _CH, _CH)
    zeros1 = jnp.zeros((n_pad, W1.shape[1]), jnp.float32)
    zeros2 = jnp.zeros((n_pad, W2.shape[1]), jnp.float32)

    h = _matmul1(x, W1)
    p1 = _sc_aggregate(h, src2, dst2, zeros1)
    h2 = _layer2_matmul(p1, W2)
    p2 = _sc_aggregate(h2, src2, dst2, zeros2)
    z, zt = _prep_z(p2, n)
    return _decoder(z, zt, n)


# speedup vs baseline: 1.1154x; 1.0047x over previous
import functools

import jax
import jax.numpy as jnp
from jax import lax
from jax.experimental import pallas as pl
from jax.experimental.pallas import tpu as pltpu
from jax.experimental.pallas import tpu_sc as plsc

_NC = 2
_NS = 16
_NW = _NC * _NS
_CH = 125


def _matmul1(x, w):
    def body(x_ref, w_ref, o_ref):
        o_ref[...] = jnp.dot(x_ref[...], w_ref[...],
                             preferred_element_type=jnp.float32)
    return pl.pallas_call(
        body,
        out_shape=jax.ShapeDtypeStruct((x.shape[0], w.shape[1]), jnp.float32),
    )(x, w)


def _sc_aggregate(h, src2, dst2, zeros):
    n, f = zeros.shape
    n_chunks = src2.shape[0]
    chunks_per_w = n_chunks // _NW
    rows_per_tile = n // _NS

    mesh = plsc.VectorSubcoreMesh(core_axis_name="c", subcore_axis_name="s")

    @functools.partial(
        pl.kernel,
        mesh=mesh,
        out_type=jax.ShapeDtypeStruct((_NC, n, f), jnp.float32),
        scratch_types=[
            pltpu.VMEM((chunks_per_w, _CH), jnp.int32),
            pltpu.VMEM((chunks_per_w, _CH), jnp.int32),
            pltpu.VMEM((_CH, f), jnp.float32),
            pltpu.VMEM((_CH, f), jnp.float32),
            pltpu.VMEM_SHARED((n, f), jnp.float32),
            pltpu.SemaphoreType.DMA,
            pltpu.SemaphoreType.DMA,
        ],
        compiler_params=pltpu.CompilerParams(use_tc_tiling_on_sc=False),
    )
    def agg(h_hbm, src_hbm, dst_hbm, z_hbm, out_hbm,
            sidx_v, didx_v, rows0_v, rows1_v, acc_sh, sem0, sem1):
        cid = lax.axis_index("c")
        sid = lax.axis_index("s")
        wid = sid * _NC + cid

        pltpu.sync_copy(z_hbm.at[pl.ds(sid * rows_per_tile, rows_per_tile)],
                        acc_sh.at[pl.ds(sid * rows_per_tile, rows_per_tile)])
        pltpu.sync_copy(src_hbm.at[pl.ds(wid * chunks_per_w, chunks_per_w)],
                        sidx_v)
        pltpu.sync_copy(dst_hbm.at[pl.ds(wid * chunks_per_w, chunks_per_w)],
                        didx_v)
        plsc.subcore_barrier()

        pairs = chunks_per_w // 2

        def start(j, rows):
            sem = sem0 if rows is rows0_v else sem1
            pltpu.async_copy(h_hbm.at[sidx_v.at[j]], rows, sem)

        def wait_add(j, rows):
            sem = sem0 if rows is rows0_v else sem1
            pltpu.make_async_copy(h_hbm.at[sidx_v.at[j]], rows, sem).wait()
            pltpu.sync_copy(rows, acc_sh.at[didx_v.at[j]], add=True)

        start(0, rows0_v)

        def step(j, carry):
            a = 2 * j
            start(a + 1, rows1_v)
            wait_add(a, rows0_v)
            start(a + 2, rows0_v)
            wait_add(a + 1, rows1_v)
            return carry

        lax.fori_loop(0, pairs - 1, step, 0)
        last = 2 * (pairs - 1)
        start(last + 1, rows1_v)
        wait_add(last, rows0_v)
        wait_add(last + 1, rows1_v)
        plsc.subcore_barrier()

        pltpu.sync_copy(acc_sh.at[pl.ds(sid * rows_per_tile, rows_per_tile)],
                        out_hbm.at[cid, pl.ds(sid * rows_per_tile,
                                              rows_per_tile)])

    return agg(h, src2, dst2, zeros)


def _layer2_matmul(partial, w):
    def body(p_ref, w_ref, o_ref):
        hidden = jnp.maximum(p_ref[0] + p_ref[1], 0.0)
        o_ref[...] = jnp.dot(hidden, w_ref[...],
                             preferred_element_type=jnp.float32)
    n_pad = partial.shape[1]
    return pl.pallas_call(
        body,
        out_shape=jax.ShapeDtypeStruct((n_pad, w.shape[1]), jnp.float32),
    )(partial, w)


def _decoder(partial, n):
    n_pad, f = partial.shape[1], partial.shape[2]
    blk = 400
    grid = n // blk

    def body(p_ref, o_ref, zt_s):
        i = pl.program_id(0)

        @pl.when(i == 0)
        def _():
            zt_s[...] = (p_ref[0, :n, :] + p_ref[1, :n, :]).T

        zb = (p_ref[0, pl.ds(i * blk, blk), :] +
              p_ref[1, pl.ds(i * blk, blk), :])
        o_ref[...] = jnp.dot(zb, zt_s[...],
                             preferred_element_type=jnp.float32)

    out2 = pl.pallas_call(
        body,
        grid=(grid,),
        in_specs=[pl.BlockSpec((2, n_pad, f), lambda i: (0, 0, 0))],
        out_specs=pl.BlockSpec((blk, n), lambda i: (i, 0)),
        out_shape=jax.ShapeDtypeStruct((n, n), jnp.float32),
        scratch_shapes=[pltpu.VMEM((f, n), jnp.float32)],
    )(partial)
    return out2.reshape(-1)


def kernel(x, edge_index, W1, W2):
    n = x.shape[0]
    e = edge_index.shape[1]
    n_pad = ((n // _NS + 7) // 8 * 8) * _NS
    src2 = edge_index[0].astype(jnp.int32).reshape(e // _CH, _CH)
    dst2 = edge_index[1].astype(jnp.int32).reshape(e // _CH, _CH)
    zeros1 = jnp.zeros((n_pad, W1.shape[1]), jnp.float32)
    zeros2 = jnp.zeros((n_pad, W2.shape[1]), jnp.float32)

    h = _matmul1(x, W1)
    p1 = _sc_aggregate(h, src2, dst2, zeros1)
    h2 = _layer2_matmul(p1, W2)
    p2 = _sc_aggregate(h2, src2, dst2, zeros2)
    return _decoder(p2, n)
